# baseline (speedup 1.0000x reference)
import jax
import jax.numpy as jnp
from jax import lax
from jax.experimental import pallas as pl
from jax.experimental.pallas import tpu as pltpu
from jax.experimental.pallas import tpu_sc as plsc

N_NODES = 100000
NUM_GRAPHS = 512
NC = 2
NS = 16
NW = NC * NS

CHUNK = 3136
LAST = N_NODES - (NW - 1) * CHUNK


def _rsqrt(sq):
  ii = plsc.bitcast(sq, jnp.int32)
  ii = 0x5F3759DF - (ii >> 1)
  r = plsc.bitcast(ii, jnp.float32)
  hs = 0.5 * sq
  r = r * (1.5 - hs * r * r)
  r = r * (1.5 - hs * r * r)
  r = r * (1.5 - hs * r * r)
  return r


def _sc_body(dip_t_hbm, batch_hbm, zero_hbm, part_hbm,
             s_v, x_v, y_v, z_v, out_v, idx_full, idx_last, acc_sh,
             sem_d, sem_i):
  c = lax.axis_index("c")
  s = lax.axis_index("s")
  wid = c * NS + s
  is_last = wid == NW - 1
  not_last = jnp.logical_not(is_last)
  base = wid * CHUNK

  def in_copies(n, idx_ref):
    cps = [
        pltpu.make_async_copy(
            dip_t_hbm.at[pl.ds(k * N_NODES + base, n)],
            vm.at[pl.ds(0, n)], sem_d)
        for k, vm in enumerate((s_v, x_v, y_v, z_v))
    ]
    cps.append(pltpu.make_async_copy(
        batch_hbm.at[pl.ds(base, n)], idx_ref, sem_i))
    return cps

  @pl.when(not_last)
  def _():
    for cp in in_copies(CHUNK, idx_full):
      cp.start()

  @pl.when(is_last)
  def _():
    for cp in in_copies(LAST, idx_last):
      cp.start()

  @pl.when(s == 0)
  def _():
    pltpu.sync_copy(zero_hbm, acc_sh)
  plsc.subcore_barrier()

  iota = lax.iota(jnp.int32, 16)
  col0 = jnp.full((16,), 0, jnp.int32)
  col1 = jnp.full((16,), 1, jnp.int32)
  col2 = jnp.full((16,), 2, jnp.int32)

  def group(g, carry):
    o = g * 16
    lanes = o + iota
    sv = plsc.load_gather(s_v, [lanes])
    xv = plsc.load_gather(x_v, [lanes])
    yv = plsc.load_gather(y_v, [lanes])
    zv = plsc.load_gather(z_v, [lanes])
    sq = xv * xv + yv * yv + zv * zv
    f = sv * _rsqrt(sq)
    rows = o + iota
    plsc.store_scatter(out_v, [rows, col0], f * xv)
    plsc.store_scatter(out_v, [rows, col1], f * yv)
    plsc.store_scatter(out_v, [rows, col2], f * zv)
    return carry

  def tile_work(n, idx_ref):
    cps = in_copies(n, idx_ref)
    for cp in cps[:4]:
      cp.wait()
    lax.fori_loop(0, n // 16, group, 0)
    cps[4].wait()
    pltpu.sync_copy(out_v.at[pl.ds(0, n)], acc_sh.at[idx_ref], add=True)

  @pl.when(not_last)
  def _():
    tile_work(CHUNK, idx_full)

  @pl.when(is_last)
  def _():
    tile_work(LAST, idx_last)

  plsc.subcore_barrier()

  @pl.when(s == 0)
  def _():
    pltpu.sync_copy(acc_sh, part_hbm.at[c])


def _sc_partials(dip_t, batch_i32, zeros):
  mesh = plsc.VectorSubcoreMesh(
      core_axis_name="c", subcore_axis_name="s", num_cores=NC,
      num_subcores=NS)
  f = pl.kernel(
      _sc_body,
      out_type=jax.ShapeDtypeStruct((NC, NUM_GRAPHS, 3), jnp.float32),
      mesh=mesh,
      compiler_params=pltpu.CompilerParams(
          needs_layout_passes=False, use_tc_tiling_on_sc=False),
      scratch_types=[
          pltpu.VMEM((CHUNK,), jnp.float32),
          pltpu.VMEM((CHUNK,), jnp.float32),
          pltpu.VMEM((CHUNK,), jnp.float32),
          pltpu.VMEM((CHUNK,), jnp.float32),
          pltpu.VMEM((CHUNK, 3), jnp.float32),
          pltpu.VMEM((CHUNK,), jnp.int32),
          pltpu.VMEM((LAST,), jnp.int32),
          pltpu.VMEM_SHARED((NUM_GRAPHS, 3), jnp.float32),
          pltpu.SemaphoreType.DMA,
          pltpu.SemaphoreType.DMA,
      ],
  )
  return f(dip_t, batch_i32, zeros)


def _finish_body(part_ref, o_ref):
  p = part_ref[0] + part_ref[1]
  o_ref[...] = jnp.sqrt(jnp.sum(p * p, axis=-1, keepdims=True))


def kernel(dipole, batch):
  dip_t = dipole.T.reshape(-1)
  batch_i32 = batch.astype(jnp.int32)
  zeros = jnp.zeros((NUM_GRAPHS, 3), jnp.float32)
  part = _sc_partials(dip_t, batch_i32, zeros)
  return pl.pallas_call(
      _finish_body,
      out_shape=jax.ShapeDtypeStruct((NUM_GRAPHS, 1), jnp.float32),
  )(part)


# speedup vs baseline: 1.0395x; 1.0395x over previous
import jax
import jax.numpy as jnp
from jax import lax
from jax.experimental import pallas as pl
from jax.experimental.pallas import tpu as pltpu
from jax.experimental.pallas import tpu_sc as plsc

N_NODES = 100000
NUM_GRAPHS = 512
NC = 2
NS = 16
NW = NC * NS

CHUNK = 3136
LAST = N_NODES - (NW - 1) * CHUNK


def _rsqrt(sq):
  ii = plsc.bitcast(sq, jnp.int32)
  ii = 0x5F3759DF - (ii >> 1)
  r = plsc.bitcast(ii, jnp.float32)
  hs = 0.5 * sq
  r = r * (1.5 - hs * r * r)
  r = r * (1.5 - hs * r * r)
  r = r * (1.5 - hs * r * r)
  return r


SEG = CHUNK // 2
SEG_L = LAST // 2


def _sc_body(dip_t_hbm, batch_hbm, zero_hbm, part_hbm,
             s_v, x_v, y_v, z_v, out_a, out_b, idx_a, idx_b,
             out_la, out_lb, idx_la, idx_lb, acc_sh,
             sem_d, sem_i, sem_sc):
  c = lax.axis_index("c")
  s = lax.axis_index("s")
  wid = c * NS + s
  is_last = wid == NW - 1
  not_last = jnp.logical_not(is_last)
  base = wid * CHUNK

  def in_copies(n, idx_refs, seg):
    cps = [
        pltpu.make_async_copy(
            dip_t_hbm.at[pl.ds(k * N_NODES + base, n)],
            vm.at[pl.ds(0, n)], sem_d)
        for k, vm in enumerate((s_v, x_v, y_v, z_v))
    ]
    icps = [
        pltpu.make_async_copy(
            batch_hbm.at[pl.ds(base + st * seg, seg)], iref, sem_i)
        for st, iref in enumerate(idx_refs)
    ]
    return cps, icps

  @pl.when(not_last)
  def _():
    cps, icps = in_copies(CHUNK, (idx_a, idx_b), SEG)
    for cp in cps + icps:
      cp.start()

  @pl.when(is_last)
  def _():
    cps, icps = in_copies(LAST, (idx_la, idx_lb), SEG_L)
    for cp in cps + icps:
      cp.start()

  @pl.when(s == 0)
  def _():
    pltpu.sync_copy(zero_hbm, acc_sh)
  plsc.subcore_barrier()

  iota = lax.iota(jnp.int32, 16)
  col0 = jnp.full((16,), 0, jnp.int32)
  col1 = jnp.full((16,), 1, jnp.int32)
  col2 = jnp.full((16,), 2, jnp.int32)

  def make_group(stage_off, out_ref):
    def group(g, carry):
      lanes = stage_off + g * 16 + iota
      sv = plsc.load_gather(s_v, [lanes])
      xv = plsc.load_gather(x_v, [lanes])
      yv = plsc.load_gather(y_v, [lanes])
      zv = plsc.load_gather(z_v, [lanes])
      sq = xv * xv + yv * yv + zv * zv
      f = sv * _rsqrt(sq)
      rows = g * 16 + iota
      plsc.store_scatter(out_ref, [rows, col0], f * xv)
      plsc.store_scatter(out_ref, [rows, col1], f * yv)
      plsc.store_scatter(out_ref, [rows, col2], f * zv)
      return carry
    return group

  def tile_work(n, out_refs, idx_refs, seg):
    cps, icps = in_copies(n, idx_refs, seg)
    for cp in cps:
      cp.wait()
    scs = []
    for st, (oref, iref) in enumerate(zip(out_refs, idx_refs)):
      icps[st].wait()
      lax.fori_loop(0, seg // 16, make_group(st * seg, oref), 0)
      scs.append(pltpu.async_copy(oref, acc_sh.at[iref], sem_sc, add=True))
    for sc_cp in scs:
      sc_cp.wait()

  @pl.when(not_last)
  def _():
    tile_work(CHUNK, (out_a, out_b), (idx_a, idx_b), SEG)

  @pl.when(is_last)
  def _():
    tile_work(LAST, (out_la, out_lb), (idx_la, idx_lb), SEG_L)

  plsc.subcore_barrier()

  @pl.when(s == 0)
  def _():
    pltpu.sync_copy(acc_sh, part_hbm.at[c])


def _sc_partials(dip_t, batch_i32, zeros):
  mesh = plsc.VectorSubcoreMesh(
      core_axis_name="c", subcore_axis_name="s", num_cores=NC,
      num_subcores=NS)
  f = pl.kernel(
      _sc_body,
      out_type=jax.ShapeDtypeStruct((NC, NUM_GRAPHS, 3), jnp.float32),
      mesh=mesh,
      compiler_params=pltpu.CompilerParams(
          needs_layout_passes=False, use_tc_tiling_on_sc=False),
      scratch_types=[
          pltpu.VMEM((CHUNK,), jnp.float32),
          pltpu.VMEM((CHUNK,), jnp.float32),
          pltpu.VMEM((CHUNK,), jnp.float32),
          pltpu.VMEM((CHUNK,), jnp.float32),
          pltpu.VMEM((SEG, 3), jnp.float32),
          pltpu.VMEM((SEG, 3), jnp.float32),
          pltpu.VMEM((SEG,), jnp.int32),
          pltpu.VMEM((SEG,), jnp.int32),
          pltpu.VMEM((SEG_L, 3), jnp.float32),
          pltpu.VMEM((SEG_L, 3), jnp.float32),
          pltpu.VMEM((SEG_L,), jnp.int32),
          pltpu.VMEM((SEG_L,), jnp.int32),
          pltpu.VMEM_SHARED((NUM_GRAPHS, 3), jnp.float32),
          pltpu.SemaphoreType.DMA,
          pltpu.SemaphoreType.DMA,
          pltpu.SemaphoreType.DMA,
      ],
  )
  return f(dip_t, batch_i32, zeros)


def _finish_body(part_ref, o_ref):
  p = part_ref[0] + part_ref[1]
  o_ref[...] = jnp.sqrt(jnp.sum(p * p, axis=-1, keepdims=True))


def kernel(dipole, batch):
  dip_t = dipole.T.reshape(-1)
  batch_i32 = batch.astype(jnp.int32)
  zeros = jnp.zeros((NUM_GRAPHS, 3), jnp.float32)
  part = _sc_partials(dip_t, batch_i32, zeros)
  return pl.pallas_call(
      _finish_body,
      out_shape=jax.ShapeDtypeStruct((NUM_GRAPHS, 1), jnp.float32),
  )(part)


# speedup vs baseline: 1.0670x; 1.0265x over previous
import jax
import jax.numpy as jnp
from jax import lax
from jax.experimental import pallas as pl
from jax.experimental.pallas import tpu as pltpu
from jax.experimental.pallas import tpu_sc as plsc

N_NODES = 100000
NUM_GRAPHS = 512
NC = 2
NS = 16
NW = NC * NS

CHUNK = 3136
LAST = N_NODES - (NW - 1) * CHUNK


def _rsqrt(sq):
  ii = plsc.bitcast(sq, jnp.int32)
  ii = 0x5F3759DF - (ii >> 1)
  r = plsc.bitcast(ii, jnp.float32)
  hs = 0.5 * sq
  r = r * (1.5 - hs * r * r)
  r = r * (1.5 - hs * r * r)
  return r


SEG = CHUNK // 2
SEG_L = LAST // 2


def _sc_body(dip_t_hbm, batch_hbm, zero_hbm, part_hbm,
             s_v, x_v, y_v, z_v, out_a, out_b, idx_a, idx_b,
             out_la, out_lb, idx_la, idx_lb, acc_sh,
             sem_d, sem_i, sem_sc):
  c = lax.axis_index("c")
  s = lax.axis_index("s")
  wid = c * NS + s
  is_last = wid == NW - 1
  not_last = jnp.logical_not(is_last)
  base = wid * CHUNK

  def in_copies(n, idx_refs, seg):
    cps = [
        pltpu.make_async_copy(
            dip_t_hbm.at[pl.ds(k * N_NODES + base, n)],
            vm.at[pl.ds(0, n)], sem_d)
        for k, vm in enumerate((s_v, x_v, y_v, z_v))
    ]
    icps = [
        pltpu.make_async_copy(
            batch_hbm.at[pl.ds(base + st * seg, seg)], iref, sem_i)
        for st, iref in enumerate(idx_refs)
    ]
    return cps, icps

  @pl.when(not_last)
  def _():
    cps, icps = in_copies(CHUNK, (idx_a, idx_b), SEG)
    for cp in cps + icps:
      cp.start()

  @pl.when(is_last)
  def _():
    cps, icps = in_copies(LAST, (idx_la, idx_lb), SEG_L)
    for cp in cps + icps:
      cp.start()

  @pl.when(s == 0)
  def _():
    pltpu.sync_copy(zero_hbm, acc_sh)
  plsc.subcore_barrier()

  iota = lax.iota(jnp.int32, 16)
  col0 = jnp.full((16,), 0, jnp.int32)
  col1 = jnp.full((16,), 1, jnp.int32)
  col2 = jnp.full((16,), 2, jnp.int32)

  def make_group(stage_off, out_ref):
    def group(g, carry):
      lanes = stage_off + g * 16 + iota
      sv = plsc.load_gather(s_v, [lanes])
      xv = plsc.load_gather(x_v, [lanes])
      yv = plsc.load_gather(y_v, [lanes])
      zv = plsc.load_gather(z_v, [lanes])
      sq = xv * xv + yv * yv + zv * zv
      f = sv * _rsqrt(sq)
      rows = g * 16 + iota
      plsc.store_scatter(out_ref, [rows, col0], f * xv)
      plsc.store_scatter(out_ref, [rows, col1], f * yv)
      plsc.store_scatter(out_ref, [rows, col2], f * zv)
      return carry
    return group

  def tile_work(n, out_refs, idx_refs, seg):
    cps, icps = in_copies(n, idx_refs, seg)
    for cp in cps:
      cp.wait()
    scs = []
    for st, (oref, iref) in enumerate(zip(out_refs, idx_refs)):
      icps[st].wait()
      lax.fori_loop(0, seg // 16, make_group(st * seg, oref), 0)
      scs.append(pltpu.async_copy(oref, acc_sh.at[iref], sem_sc, add=True))
    for sc_cp in scs:
      sc_cp.wait()

  @pl.when(not_last)
  def _():
    tile_work(CHUNK, (out_a, out_b), (idx_a, idx_b), SEG)

  @pl.when(is_last)
  def _():
    tile_work(LAST, (out_la, out_lb), (idx_la, idx_lb), SEG_L)

  plsc.subcore_barrier()

  @pl.when(s == 0)
  def _():
    pltpu.sync_copy(acc_sh, part_hbm.at[c])


def _sc_partials(dip_t, batch_i32, zeros):
  mesh = plsc.VectorSubcoreMesh(
      core_axis_name="c", subcore_axis_name="s", num_cores=NC,
      num_subcores=NS)
  f = pl.kernel(
      _sc_body,
      out_type=jax.ShapeDtypeStruct((NC, NUM_GRAPHS, 3), jnp.float32),
      mesh=mesh,
      compiler_params=pltpu.CompilerParams(
          needs_layout_passes=False, use_tc_tiling_on_sc=False),
      scratch_types=[
          pltpu.VMEM((CHUNK,), jnp.float32),
          pltpu.VMEM((CHUNK,), jnp.float32),
          pltpu.VMEM((CHUNK,), jnp.float32),
          pltpu.VMEM((CHUNK,), jnp.float32),
          pltpu.VMEM((SEG, 3), jnp.float32),
          pltpu.VMEM((SEG, 3), jnp.float32),
          pltpu.VMEM((SEG,), jnp.int32),
          pltpu.VMEM((SEG,), jnp.int32),
          pltpu.VMEM((SEG_L, 3), jnp.float32),
          pltpu.VMEM((SEG_L, 3), jnp.float32),
          pltpu.VMEM((SEG_L,), jnp.int32),
          pltpu.VMEM((SEG_L,), jnp.int32),
          pltpu.VMEM_SHARED((NUM_GRAPHS, 3), jnp.float32),
          pltpu.SemaphoreType.DMA,
          pltpu.SemaphoreType.DMA,
          pltpu.SemaphoreType.DMA,
      ],
  )
  return f(dip_t, batch_i32, zeros)


def _finish_body(part_ref, o_ref):
  p = part_ref[0] + part_ref[1]
  o_ref[...] = jnp.sqrt(jnp.sum(p * p, axis=-1, keepdims=True))


def kernel(dipole, batch):
  dip_t = dipole.T.reshape(-1)
  batch_i32 = batch.astype(jnp.int32)
  zeros = jnp.zeros((NUM_GRAPHS, 3), jnp.float32)
  part = _sc_partials(dip_t, batch_i32, zeros)
  return pl.pallas_call(
      _finish_body,
      out_shape=jax.ShapeDtypeStruct((NUM_GRAPHS, 1), jnp.float32),
  )(part)
